# Initial kernel scaffold; baseline (speedup 1.0000x reference)
import functools

import jax
import jax.numpy as jnp
from jax import lax
from jax.experimental import pallas as pl
from jax.experimental.pallas import tpu as pltpu
from jax.experimental.pallas import tpu_sc as plsc

NUM_BUCKETS = 100000
NUM_HEADS = 4
HEAD_DIM = 32
STEP = 9

ROWS = 16384
COLS = 26
N = ROWS * COLS

NC = 2
NS = 16
NW = NC * NS
PER_W = N // NW
CHUNK = 128
NCHUNK = PER_W // CHUNK
LANES = 16


def _body(idx_hbm, w0, w1, w2, w3, out_hbm, idx_v, hidx_v, rows_v, sem):
    tables = (w0, w1, w2, w3)
    wid = lax.axis_index("s") * NC + lax.axis_index("c")
    wbase = wid * PER_W

    def chunk_body(j, carry):
        base = wbase + j * CHUNK
        pltpu.sync_copy(idx_hbm.at[pl.ds(base, CHUNK)], idx_v)
        for i in range(CHUNK // LANES):
            sl = pl.ds(i * LANES, LANES)
            x = idx_v[sl]
            for h in range(1, NUM_HEADS):
                x = jnp.where(x >= STEP, x - STEP, x + (NUM_BUCKETS - STEP))
                hidx_v[h - 1, sl] = x
        copies = [pltpu.async_copy(tables[0].at[idx_v], rows_v.at[0], sem)]
        for h in range(1, NUM_HEADS):
            copies.append(
                pltpu.async_copy(tables[h].at[hidx_v.at[h - 1]], rows_v.at[h], sem)
            )
        for c in copies:
            c.wait()
        for h in range(NUM_HEADS):
            pltpu.sync_copy(
                rows_v.at[h],
                out_hbm.at[pl.ds(base, CHUNK), pl.ds(h * HEAD_DIM, HEAD_DIM)],
            )
        return carry

    lax.fori_loop(0, NCHUNK, chunk_body, 0)


def kernel(hashed_value, W0, W1, W2, W3):
    idx_flat = hashed_value.reshape(N).astype(jnp.int32)
    mesh = plsc.VectorSubcoreMesh(
        core_axis_name="c", subcore_axis_name="s", num_cores=NC, num_subcores=NS
    )
    run = pl.kernel(
        _body,
        out_type=jax.ShapeDtypeStruct((N, NUM_HEADS * HEAD_DIM), jnp.float32),
        mesh=mesh,
        scratch_types=[
            pltpu.VMEM((CHUNK,), jnp.int32),
            pltpu.VMEM((NUM_HEADS - 1, CHUNK), jnp.int32),
            pltpu.VMEM((NUM_HEADS, CHUNK, HEAD_DIM), jnp.float32),
            pltpu.SemaphoreType.DMA,
        ],
    )
    out = run(idx_flat, W0, W1, W2, W3)
    return out.reshape(ROWS, COLS, NUM_HEADS * HEAD_DIM)


# speedup vs baseline: 9.9303x; 9.9303x over previous
import functools

import jax
import jax.numpy as jnp
from jax import lax
from jax.experimental import pallas as pl
from jax.experimental.pallas import tpu as pltpu
from jax.experimental.pallas import tpu_sc as plsc

NUM_BUCKETS = 100000
NUM_HEADS = 4
HEAD_DIM = 32
STEP = 9

ROWS = 16384
COLS = 26
N = ROWS * COLS

NC = 2
NS = 16
NW = NC * NS
PER_W = N // NW
CHUNK = 128
NCHUNK = PER_W // CHUNK
LANES = 16


def _body(idx_hbm, w0, w1, w2, w3, out_hbm, idx_v, hidx_v, rows_v, sem):
    tables = (w0, w1, w2, w3)
    wid = lax.axis_index("s") * NC + lax.axis_index("c")
    wbase = wid * PER_W

    def chunk_body(j, carry):
        base = wbase + j * CHUNK
        pltpu.sync_copy(idx_hbm.at[pl.ds(base, CHUNK)], idx_v)
        for i in range(CHUNK // LANES):
            sl = pl.ds(i * LANES, LANES)
            x = idx_v[sl]
            for h in range(1, NUM_HEADS):
                x = jnp.where(x >= STEP, x - STEP, x + (NUM_BUCKETS - STEP))
                hidx_v[h - 1, sl] = x
        copies = [pltpu.async_copy(tables[0].at[idx_v], rows_v.at[0], sem)]
        for h in range(1, NUM_HEADS):
            copies.append(
                pltpu.async_copy(tables[h].at[hidx_v.at[h - 1]], rows_v.at[h], sem)
            )
        for c in copies:
            c.wait()
        for h in range(NUM_HEADS):
            pltpu.sync_copy(
                rows_v.at[h],
                out_hbm.at[pl.ds(base, CHUNK), pl.ds(h * HEAD_DIM, HEAD_DIM)],
            )
        return carry

    lax.fori_loop(0, NCHUNK, chunk_body, 0)


def kernel(hashed_value, W0, W1, W2, W3):
    idx_flat = hashed_value.reshape(N).astype(jnp.int32)
    mesh = plsc.VectorSubcoreMesh(
        core_axis_name="c", subcore_axis_name="s", num_cores=NC, num_subcores=NS
    )
    run = pl.kernel(
        _body,
        out_type=jax.ShapeDtypeStruct((N, NUM_HEADS * HEAD_DIM), jnp.float32),
        mesh=mesh,
        scratch_types=[
            pltpu.VMEM((CHUNK,), jnp.int32),
            pltpu.VMEM((NUM_HEADS - 1, CHUNK), jnp.int32),
            pltpu.VMEM((NUM_HEADS, CHUNK, HEAD_DIM), jnp.float32),
            pltpu.SemaphoreType.DMA,
        ],
        compiler_params=pltpu.CompilerParams(use_tc_tiling_on_sc=False),
    )
    out = run(idx_flat, W0, W1, W2, W3)
    return out.reshape(ROWS, COLS, NUM_HEADS * HEAD_DIM)


# speedup vs baseline: 11.7317x; 1.1814x over previous
import jax
import jax.numpy as jnp
from jax import lax
from jax.experimental import pallas as pl
from jax.experimental.pallas import tpu as pltpu
from jax.experimental.pallas import tpu_sc as plsc

NUM_BUCKETS = 100000
NUM_HEADS = 4
HEAD_DIM = 32
STEP = 9

ROWS = 16384
COLS = 26
N = ROWS * COLS

NC = 2
NS = 16
NW = NC * NS
PER_W = N // NW
CHUNK = 128
NCHUNK = PER_W // CHUNK
LANES = 16
K = 4


def _body(idx_hbm, w0, w1, w2, w3, out_hbm, raw_v, hidx_v, rows_v, *sems):
    gsem = sems[:K]
    wsem = sems[K:]
    tables = (w0, w1, w2, w3)
    wid = lax.axis_index("s") * NC + lax.axis_index("c")
    wbase = wid * PER_W

    pltpu.sync_copy(idx_hbm.at[pl.ds(wid * NCHUNK, NCHUNK)], raw_v)

    def fire_g(c, slot):
        for i in range(CHUNK // LANES):
            sl = pl.ds(i * LANES, LANES)
            x = raw_v[c, sl]
            for h in range(1, NUM_HEADS):
                x = jnp.where(x >= STEP, x - STEP, x + (NUM_BUCKETS - STEP))
                hidx_v[slot, h - 1, sl] = x
        pltpu.async_copy(tables[0].at[raw_v.at[c]], rows_v.at[slot, 0], gsem[slot])
        for h in range(1, NUM_HEADS):
            pltpu.async_copy(
                tables[h].at[hidx_v.at[slot, h - 1]], rows_v.at[slot, h], gsem[slot]
            )

    def wait_g(slot):
        for h in range(NUM_HEADS):
            pltpu.make_async_copy(
                tables[h].at[raw_v.at[0]], rows_v.at[slot, h], gsem[slot]
            ).wait()

    def fire_w(c, slot):
        base = wbase + c * CHUNK
        for h in range(NUM_HEADS):
            pltpu.async_copy(
                rows_v.at[slot, h],
                out_hbm.at[pl.ds(base, CHUNK), pl.ds(h * HEAD_DIM, HEAD_DIM)],
                wsem[slot],
            )

    def wait_w(slot):
        for h in range(NUM_HEADS):
            pltpu.make_async_copy(
                rows_v.at[slot, h],
                out_hbm.at[pl.ds(0, CHUNK), pl.ds(h * HEAD_DIM, HEAD_DIM)],
                wsem[slot],
            ).wait()

    fire_g(0, 0)
    fire_g(1, 1)
    wait_g(0)
    fire_w(0, 0)
    fire_g(2, 2)
    wait_g(1)
    fire_w(1, 1)
    fire_g(3, 3)

    def main_body(t, carry):
        for b in range(K):
            j = 2 + K * t + b
            s_a = (2 + b) % K
            wait_g(s_a)
            fire_w(j, s_a)
            wait_w(b)
            fire_g(j + 2, b)
        return carry

    lax.fori_loop(0, (NCHUNK - K) // K, main_body, 0)

    wait_g((NCHUNK - 2) % K)
    fire_w(NCHUNK - 2, (NCHUNK - 2) % K)
    wait_w((NCHUNK - 4) % K)
    wait_g((NCHUNK - 1) % K)
    fire_w(NCHUNK - 1, (NCHUNK - 1) % K)
    wait_w((NCHUNK - 3) % K)
    wait_w((NCHUNK - 2) % K)
    wait_w((NCHUNK - 1) % K)


def kernel(hashed_value, W0, W1, W2, W3):
    idx_2d = hashed_value.reshape(N // CHUNK, CHUNK).astype(jnp.int32)
    mesh = plsc.VectorSubcoreMesh(
        core_axis_name="c", subcore_axis_name="s", num_cores=NC, num_subcores=NS
    )
    run = pl.kernel(
        _body,
        out_type=jax.ShapeDtypeStruct((N, NUM_HEADS * HEAD_DIM), jnp.float32),
        mesh=mesh,
        scratch_types=(
            [
                pltpu.VMEM((NCHUNK, CHUNK), jnp.int32),
                pltpu.VMEM((K, NUM_HEADS - 1, CHUNK), jnp.int32),
                pltpu.VMEM((K, NUM_HEADS, CHUNK, HEAD_DIM), jnp.float32),
            ]
            + [pltpu.SemaphoreType.DMA] * (2 * K)
        ),
        compiler_params=pltpu.CompilerParams(use_tc_tiling_on_sc=False),
    )
    out = run(idx_2d, W0, W1, W2, W3)
    return out.reshape(ROWS, COLS, NUM_HEADS * HEAD_DIM)


# speedup vs baseline: 26.5937x; 2.2668x over previous
import jax
import jax.numpy as jnp
from jax import lax
from jax.experimental import pallas as pl
from jax.experimental.pallas import tpu as pltpu
from jax.experimental.pallas import tpu_sc as plsc

NUM_BUCKETS = 100000
NUM_HEADS = 4
HEAD_DIM = 32
STEP = 9

ROWS = 16384
COLS = 26
N = ROWS * COLS

NC = 2
NS = 16
NW = NC * NS
PER_W = N // NW
CHUNK = 128
NCHUNK = PER_W // CHUNK
LANES = 16
K = 4


def _body(idx_hbm, w0, w1, w2, w3, out_hbm, raw_v, hidx_v, rows_v, *sems):
    gsem = sems[:K]
    wsem = sems[K:]
    tables = (w0, w1, w2, w3)
    wid = lax.axis_index("s") * NC + lax.axis_index("c")
    wbase = wid * PER_W

    pltpu.sync_copy(idx_hbm.at[pl.ds(wid * NCHUNK, NCHUNK)], raw_v)

    def fire_g(c, slot):
        for i in range(CHUNK // LANES):
            sl = pl.ds(i * LANES, LANES)
            x = raw_v[c, sl]
            for h in range(1, NUM_HEADS):
                x = jnp.where(x >= STEP, x - STEP, x + (NUM_BUCKETS - STEP))
                hidx_v[slot, h - 1, sl] = x
        pltpu.async_copy(tables[0].at[raw_v.at[c]], rows_v.at[slot, 0], gsem[slot])
        for h in range(1, NUM_HEADS):
            pltpu.async_copy(
                tables[h].at[hidx_v.at[slot, h - 1]], rows_v.at[slot, h], gsem[slot]
            )

    def wait_g(slot):
        for h in range(NUM_HEADS):
            pltpu.make_async_copy(
                tables[h].at[raw_v.at[0]], rows_v.at[slot, h], gsem[slot]
            ).wait()

    def fire_w(c, slot):
        base = wbase + c * CHUNK
        for h in range(NUM_HEADS):
            pltpu.async_copy(
                rows_v.at[slot, h],
                out_hbm.at[pl.ds(base, CHUNK), pl.ds(h * HEAD_DIM, HEAD_DIM)],
                wsem[slot],
            )

    def wait_w(slot):
        for h in range(NUM_HEADS):
            pltpu.make_async_copy(
                rows_v.at[slot, h],
                out_hbm.at[pl.ds(0, CHUNK), pl.ds(h * HEAD_DIM, HEAD_DIM)],
                wsem[slot],
            ).wait()

    fire_g(0, 0)
    fire_g(1, 1)
    wait_g(0)
    fire_w(0, 0)
    fire_g(2, 2)
    wait_g(1)
    fire_w(1, 1)
    fire_g(3, 3)

    def main_body(t, carry):
        for b in range(K):
            j = 2 + K * t + b
            s_a = (2 + b) % K
            wait_g(s_a)
            fire_w(j, s_a)
            wait_w(b)
            fire_g(j + 2, b)
        return carry

    lax.fori_loop(0, (NCHUNK - K) // K, main_body, 0)

    wait_g((NCHUNK - 2) % K)
    fire_w(NCHUNK - 2, (NCHUNK - 2) % K)
    wait_w((NCHUNK - 4) % K)
    wait_g((NCHUNK - 1) % K)
    fire_w(NCHUNK - 1, (NCHUNK - 1) % K)
    wait_w((NCHUNK - 3) % K)
    wait_w((NCHUNK - 2) % K)
    wait_w((NCHUNK - 1) % K)


def kernel(hashed_value, W0, W1, W2, W3):
    idx_2d = hashed_value.T.reshape(N // CHUNK, CHUNK).astype(jnp.int32)
    mesh = plsc.VectorSubcoreMesh(
        core_axis_name="c", subcore_axis_name="s", num_cores=NC, num_subcores=NS
    )
    run = pl.kernel(
        _body,
        out_type=jax.ShapeDtypeStruct((N, NUM_HEADS * HEAD_DIM), jnp.float32),
        mesh=mesh,
        scratch_types=(
            [
                pltpu.VMEM((NCHUNK, CHUNK), jnp.int32),
                pltpu.VMEM((K, NUM_HEADS - 1, CHUNK), jnp.int32),
                pltpu.VMEM((K, NUM_HEADS, CHUNK, HEAD_DIM), jnp.float32),
            ]
            + [pltpu.SemaphoreType.DMA] * (2 * K)
        ),
        compiler_params=pltpu.CompilerParams(use_tc_tiling_on_sc=False),
    )
    out = run(idx_2d, W0, W1, W2, W3)
    return out.reshape(COLS, ROWS, NUM_HEADS * HEAD_DIM).transpose(1, 0, 2)


# speedup vs baseline: 26.6259x; 1.0012x over previous
import jax
import jax.numpy as jnp
from jax import lax
from jax.experimental import pallas as pl
from jax.experimental.pallas import tpu as pltpu
from jax.experimental.pallas import tpu_sc as plsc

NUM_BUCKETS = 100000
NUM_HEADS = 4
HEAD_DIM = 32
STEP = 9

ROWS = 16384
COLS = 26
N = ROWS * COLS

NC = 2
NS = 16
NW = NC * NS
PER_W = N // NW
CHUNK = 128
NCHUNK = PER_W // CHUNK
LANES = 16
K = 4
SLOT_R = NUM_HEADS * CHUNK


def _body(idx_hbm, w0, w1, w2, w3, out_hbm, raw_v, hidx_v, rows_v, *sems):
    gsem = sems[:K]
    wsem = sems[K:]
    tables = (w0, w1, w2, w3)
    wid = lax.axis_index("s") * NC + lax.axis_index("c")
    wbase = wid * PER_W

    pltpu.sync_copy(idx_hbm.at[pl.ds(wid * NCHUNK, NCHUNK)], raw_v)

    def fire_g(c, slot):
        for i in range(CHUNK // LANES):
            sl = pl.ds(i * LANES, LANES)
            x = raw_v[c, sl]
            for h in range(1, NUM_HEADS):
                x = jnp.where(x >= STEP, x - STEP, x + (NUM_BUCKETS - STEP))
                hidx_v[slot, h - 1, sl] = x
        pltpu.async_copy(
            tables[0].at[raw_v.at[c]],
            rows_v.at[slot, pl.ds(0, CHUNK), :],
            gsem[slot],
        )
        for h in range(1, NUM_HEADS):
            pltpu.async_copy(
                tables[h].at[hidx_v.at[slot, h - 1]],
                rows_v.at[slot, pl.ds(h * CHUNK, CHUNK), :],
                gsem[slot],
            )

    def wait_g(slot):
        pltpu.make_async_copy(
            out_hbm.at[pl.ds(0, SLOT_R), pl.ds(0, HEAD_DIM)],
            rows_v.at[slot],
            gsem[slot],
        ).wait()

    def fire_w(c, slot):
        base = wbase + c * CHUNK
        for h in range(NUM_HEADS):
            pltpu.async_copy(
                rows_v.at[slot, pl.ds(h * CHUNK, CHUNK), :],
                out_hbm.at[pl.ds(base, CHUNK), pl.ds(h * HEAD_DIM, HEAD_DIM)],
                wsem[slot],
            )

    def wait_w(slot):
        pltpu.make_async_copy(
            rows_v.at[slot],
            out_hbm.at[pl.ds(0, SLOT_R), pl.ds(0, HEAD_DIM)],
            wsem[slot],
        ).wait()

    fire_g(0, 0)
    fire_g(1, 1)
    wait_g(0)
    fire_w(0, 0)
    fire_g(2, 2)
    wait_g(1)
    fire_w(1, 1)
    fire_g(3, 3)

    def main_body(t, carry):
        for b in range(K):
            j = 2 + K * t + b
            s_a = (2 + b) % K
            wait_g(s_a)
            fire_w(j, s_a)
            wait_w(b)
            fire_g(j + 2, b)
        return carry

    lax.fori_loop(0, (NCHUNK - K) // K, main_body, 0)

    wait_g((NCHUNK - 2) % K)
    fire_w(NCHUNK - 2, (NCHUNK - 2) % K)
    wait_w((NCHUNK - 4) % K)
    wait_g((NCHUNK - 1) % K)
    fire_w(NCHUNK - 1, (NCHUNK - 1) % K)
    wait_w((NCHUNK - 3) % K)
    wait_w((NCHUNK - 2) % K)
    wait_w((NCHUNK - 1) % K)


def kernel(hashed_value, W0, W1, W2, W3):
    idx_2d = hashed_value.T.reshape(N // CHUNK, CHUNK).astype(jnp.int32)
    mesh = plsc.VectorSubcoreMesh(
        core_axis_name="c", subcore_axis_name="s", num_cores=NC, num_subcores=NS
    )
    run = pl.kernel(
        _body,
        out_type=jax.ShapeDtypeStruct((N, NUM_HEADS * HEAD_DIM), jnp.float32),
        mesh=mesh,
        scratch_types=(
            [
                pltpu.VMEM((NCHUNK, CHUNK), jnp.int32),
                pltpu.VMEM((K, NUM_HEADS - 1, CHUNK), jnp.int32),
                pltpu.VMEM((K, SLOT_R, HEAD_DIM), jnp.float32),
            ]
            + [pltpu.SemaphoreType.DMA] * (2 * K)
        ),
        compiler_params=pltpu.CompilerParams(use_tc_tiling_on_sc=False),
    )
    out = run(idx_2d, W0, W1, W2, W3)
    return out.reshape(COLS, ROWS, NUM_HEADS * HEAD_DIM).transpose(1, 0, 2)


# speedup vs baseline: 26.9841x; 1.0134x over previous
import jax
import jax.numpy as jnp
from jax import lax
from jax.experimental import pallas as pl
from jax.experimental.pallas import tpu as pltpu
from jax.experimental.pallas import tpu_sc as plsc

NUM_BUCKETS = 100000
NUM_HEADS = 4
HEAD_DIM = 32
STEP = 9

ROWS = 16384
COLS = 26
N = ROWS * COLS

NC = 2
NS = 16
NW = NC * NS
PER_W = N // NW
CHUNK = 128
NCHUNK = PER_W // CHUNK
LANES = 16
K = 8


def _make_body(head, writes_output):
    def body(*args):
        if writes_output:
            idx_hbm, w, out_hbm = args[:3]
        else:
            out_hbm, idx_hbm, w = args[:3]
        raw_v, hidx_v, rows_v = args[3:6]
        gsem = args[6:6 + K]
        wsem = args[6 + K:6 + 2 * K]
        wid = lax.axis_index("s") * NC + lax.axis_index("c")
        wbase = wid * PER_W
        band = pl.ds(head * HEAD_DIM, HEAD_DIM)

        pltpu.sync_copy(idx_hbm.at[pl.ds(wid * NCHUNK, NCHUNK)], raw_v)

        def fire_g(c, slot):
            if head == 0:
                pltpu.async_copy(w.at[raw_v.at[c]], rows_v.at[slot], gsem[slot])
                return
            for i in range(CHUNK // LANES):
                sl = pl.ds(i * LANES, LANES)
                x = raw_v[c, sl]
                for _ in range(head):
                    x = jnp.where(x >= STEP, x - STEP, x + (NUM_BUCKETS - STEP))
                hidx_v[slot, sl] = x
            pltpu.async_copy(w.at[hidx_v.at[slot]], rows_v.at[slot], gsem[slot])

        def wait_g(slot):
            pltpu.make_async_copy(
                w.at[raw_v.at[0]], rows_v.at[slot], gsem[slot]
            ).wait()

        def fire_w(c, slot):
            base = wbase + c * CHUNK
            pltpu.async_copy(
                rows_v.at[slot], out_hbm.at[pl.ds(base, CHUNK), band], wsem[slot]
            )

        def wait_w(slot):
            pltpu.make_async_copy(
                rows_v.at[slot], out_hbm.at[pl.ds(0, CHUNK), band], wsem[slot]
            ).wait()

        D = K - 2
        for c in range(D):
            fire_g(c, c)
        for j in range(2):
            wait_g(j % K)
            fire_w(j, j % K)
            fire_g(j + D, (j + D) % K)

        def main_body(t, carry):
            for b in range(K):
                j = 2 + K * t + b
                s_a = (2 + b) % K
                wait_g(s_a)
                fire_w(j, s_a)
                wait_w(b)
                fire_g(j + D, b)
            return carry

        lax.fori_loop(0, (NCHUNK - D - 2) // K, main_body, 0)

        for j in range(NCHUNK - D, NCHUNK):
            wait_g(j % K)
            fire_w(j, j % K)
            wait_w((j - 2) % K)
        wait_w((NCHUNK - 2) % K)
        wait_w((NCHUNK - 1) % K)

    return body


def kernel(hashed_value, W0, W1, W2, W3):
    idx_2d = hashed_value.T.reshape(N // CHUNK, CHUNK).astype(jnp.int32)
    mesh = plsc.VectorSubcoreMesh(
        core_axis_name="c", subcore_axis_name="s", num_cores=NC, num_subcores=NS
    )
    params = pltpu.CompilerParams(use_tc_tiling_on_sc=False)

    def scratch(head):
        return (
            [
                pltpu.VMEM((NCHUNK, CHUNK), jnp.int32),
                pltpu.VMEM((K, CHUNK), jnp.int32),
                pltpu.VMEM((K, CHUNK, HEAD_DIM), jnp.float32),
            ]
            + [pltpu.SemaphoreType.DMA] * (2 * K)
        )

    k0 = pl.kernel(
        _make_body(0, writes_output=True),
        out_type=jax.ShapeDtypeStruct((N, NUM_HEADS * HEAD_DIM), jnp.float32),
        mesh=mesh,
        scratch_types=scratch(0),
        compiler_params=params,
    )
    out0 = k0(idx_2d, W0)
    o_ref = jax.new_ref(out0)
    for h, w in ((1, W1), (2, W2), (3, W3)):
        kh = pl.kernel(
            _make_body(h, writes_output=False),
            out_type=(),
            mesh=mesh,
            scratch_types=scratch(h),
            compiler_params=params,
        )
        kh(o_ref, idx_2d, w)
    out = o_ref[...]
    return out.reshape(COLS, ROWS, NUM_HEADS * HEAD_DIM).transpose(1, 0, 2)
